# Initial kernel scaffold; baseline (speedup 1.0000x reference)
import jax
import jax.numpy as jnp
from jax.experimental import pallas as pl


def kernel(token_types, segments, semantic_embeds, categories, token_type_table, segment_table, cat_tables, pe):
    raise NotImplementedError("write your pallas kernel here")


# speedup vs baseline: 7.0079x; 7.0079x over previous
import functools

import jax
import jax.numpy as jnp
from jax import lax
from jax.experimental import pallas as pl
from jax.experimental.pallas import tpu as pltpu
from jax.experimental.pallas import tpu_sc as plsc

_B, _L, _D = 1024, 200, 64
_N = _B * _L
_NC, _NS = 2, 16
_NW = _NC * _NS
_ROWS = 128
_PER_W = _N // _NW
_K = _PER_W // _ROWS
_NIDX = 6


def _embed_body(idx_hbm, big_hbm, small_hbm, out_hbm, idx_v, acc_v, sem):
    w = lax.axis_index("s") * _NC + lax.axis_index("c")

    def chunk(k, carry):
        pltpu.sync_copy(idx_hbm.at[w, k], idx_v)
        pltpu.async_copy(big_hbm.at[idx_v.at[0]], acc_v, sem).wait()
        cps = [
            pltpu.async_copy(big_hbm.at[idx_v.at[j]], acc_v, sem, add=True)
            for j in range(1, 4)
        ]
        cps += [
            pltpu.async_copy(small_hbm.at[idx_v.at[j]], acc_v, sem, add=True)
            for j in range(4, 6)
        ]
        for cp in cps:
            cp.wait()
        base = (w * _K + k) * _ROWS
        pltpu.sync_copy(acc_v, out_hbm.at[pl.ds(base, _ROWS)])
        return carry

    lax.fori_loop(0, _K, chunk, 0)


_embed = functools.partial(
    pl.kernel,
    out_type=jax.ShapeDtypeStruct((_N, _D), jnp.float32),
    mesh=plsc.VectorSubcoreMesh(core_axis_name="c", subcore_axis_name="s"),
    scratch_types=[
        pltpu.VMEM((_NIDX, _ROWS), jnp.int32),
        pltpu.VMEM((_ROWS, _D), jnp.float32),
        pltpu.SemaphoreType.DMA,
    ],
    compiler_params=pltpu.CompilerParams(use_tc_tiling_on_sc=False),
)(_embed_body)


def kernel(token_types, segments, semantic_embeds, categories,
           token_type_table, segment_table, cat_tables, pe):
    del semantic_embeds
    T, V, _ = cat_tables.shape
    n_seg = segment_table.shape[0]
    big = cat_tables.reshape(T * V, _D)
    fused_small = (token_type_table[:, None, :]
                   + segment_table[None, :, :]).reshape(-1, _D)
    n_fused = fused_small.shape[0]
    small = jnp.concatenate([fused_small, pe[0]], axis=0)

    cat_idx = (categories.reshape(_N, T).astype(jnp.int32)
               + jnp.arange(T, dtype=jnp.int32) * V)
    fused_idx = (token_types.reshape(_N).astype(jnp.int32) * n_seg
                 + segments.reshape(_N).astype(jnp.int32))
    pos_idx = n_fused + jnp.broadcast_to(
        jnp.arange(_L, dtype=jnp.int32), (_B, _L)).reshape(_N)
    idx6 = jnp.concatenate(
        [cat_idx.T, fused_idx[None], pos_idx[None]], axis=0)
    idx_all = idx6.reshape(_NIDX, _NW, _K, _ROWS).transpose(1, 2, 0, 3)

    out = _embed(idx_all, big, small)
    return out.reshape(_B, _L, _D)


# speedup vs baseline: 7.1993x; 1.0273x over previous
import functools

import jax
import jax.numpy as jnp
from jax import lax
from jax.experimental import pallas as pl
from jax.experimental.pallas import tpu as pltpu
from jax.experimental.pallas import tpu_sc as plsc

_B, _L, _D = 1024, 200, 64
_N = _B * _L
_NC, _NS = 2, 16
_NW = _NC * _NS
_ROWS = 128
_PER_W = _N // _NW
_K = _PER_W // _ROWS
_NIDX = 6
_NBUF = 5
_G = _K // _NBUF


def _embed_body(idx_hbm, big_hbm, small_hbm, out_hbm,
                idx_v, acc_v, sem_g0, sem_add, sem_wb):
    w = lax.axis_index("s") * _NC + lax.axis_index("c")
    pltpu.sync_copy(idx_hbm.at[w], idx_v)

    def out_slice(k):
        return out_hbm.at[pl.ds((w * _K + k) * _ROWS, _ROWS)]

    def fire_g0(k, b):
        pltpu.async_copy(big_hbm.at[idx_v.at[k, 0]], acc_v.at[b], sem_g0.at[b])

    def fire_adds(k, b):
        pltpu.make_async_copy(
            big_hbm.at[idx_v.at[k, 0]], acc_v.at[b], sem_g0.at[b]).wait()
        for j in range(1, 4):
            pltpu.async_copy(big_hbm.at[idx_v.at[k, j]], acc_v.at[b],
                             sem_add.at[b], add=True)
        for j in range(4, _NIDX):
            pltpu.async_copy(small_hbm.at[idx_v.at[k, j]], acc_v.at[b],
                             sem_add.at[b], add=True)

    def fire_wb(k, b):
        for _ in range(_NIDX - 1):
            pltpu.make_async_copy(
                big_hbm.at[idx_v.at[k, 1]], acc_v.at[b], sem_add.at[b]).wait()
        pltpu.async_copy(acc_v.at[b], out_slice(k), sem_wb.at[b])

    def wait_wb(k, b):
        pltpu.make_async_copy(acc_v.at[b], out_slice(k), sem_wb.at[b]).wait()

    for b in range(_NBUF):
        fire_g0(b, b)
    for b in range(_NBUF):
        fire_adds(b, b)

    def outer(g, carry):
        for b in range(_NBUF):
            fire_wb((g - 1) * _NBUF + b, b)
        for b in range(_NBUF):
            wait_wb((g - 1) * _NBUF + b, b)
            fire_g0(g * _NBUF + b, b)
        for b in range(_NBUF):
            fire_adds(g * _NBUF + b, b)
        return carry

    lax.fori_loop(1, _G, outer, 0)

    for b in range(_NBUF):
        fire_wb((_G - 1) * _NBUF + b, b)
    for b in range(_NBUF):
        wait_wb((_G - 1) * _NBUF + b, b)


_embed = functools.partial(
    pl.kernel,
    out_type=jax.ShapeDtypeStruct((_N, _D), jnp.float32),
    mesh=plsc.VectorSubcoreMesh(core_axis_name="c", subcore_axis_name="s"),
    scratch_types=[
        pltpu.VMEM((_K, _NIDX, _ROWS), jnp.int32),
        pltpu.VMEM((_NBUF, _ROWS, _D), jnp.float32),
        pltpu.SemaphoreType.DMA((_NBUF,)),
        pltpu.SemaphoreType.DMA((_NBUF,)),
        pltpu.SemaphoreType.DMA((_NBUF,)),
    ],
    compiler_params=pltpu.CompilerParams(use_tc_tiling_on_sc=False),
)(_embed_body)


def kernel(token_types, segments, semantic_embeds, categories,
           token_type_table, segment_table, cat_tables, pe):
    del semantic_embeds
    T, V, _ = cat_tables.shape
    n_seg = segment_table.shape[0]
    big = cat_tables.reshape(T * V, _D)
    fused_small = (token_type_table[:, None, :]
                   + segment_table[None, :, :]).reshape(-1, _D)
    n_fused = fused_small.shape[0]
    small = jnp.concatenate([fused_small, pe[0]], axis=0)

    cat_idx = (categories.reshape(_N, T).astype(jnp.int32)
               + jnp.arange(T, dtype=jnp.int32) * V)
    fused_idx = (token_types.reshape(_N).astype(jnp.int32) * n_seg
                 + segments.reshape(_N).astype(jnp.int32))
    pos_idx = n_fused + jnp.broadcast_to(
        jnp.arange(_L, dtype=jnp.int32), (_B, _L)).reshape(_N)
    idx6 = jnp.concatenate(
        [cat_idx.T, fused_idx[None], pos_idx[None]], axis=0)
    idx_all = idx6.reshape(_NIDX, _NW, _K, _ROWS).transpose(1, 2, 0, 3)

    out = _embed(idx_all, big, small)
    return out.reshape(_B, _L, _D)
